# baseline (speedup 1.0000x reference)
import functools
import math

import jax
import jax.numpy as jnp
from jax import lax
from jax.experimental import pallas as pl
from jax.experimental.pallas import tpu as pltpu
from jax.experimental.pallas import tpu_sc as plsc

VOCAB = 1000000
MAXLEN = 200
EDIM = 32
BATCH = 4096

TOTAL_ROWS = BATCH * MAXLEN
GATHER = 128
CHUNK = 2
UROWS = GATHER * CHUNK
NBUF = 6
GLAG = 4

POS_REPS = (MAXLEN + UROWS - 1) // MAXLEN + 1


def _build_sc_kernel():
    info = plsc.get_sparse_core_info()
    nc, ns = info.num_cores, info.num_subcores
    nw = nc * ns
    rows_per_w = TOTAL_ROWS // nw
    units = rows_per_w // UROWS
    padded_steps = ((units + GLAG + NBUF - 1) // NBUF) * NBUF

    mesh = plsc.VectorSubcoreMesh(core_axis_name="c", subcore_axis_name="s")

    @functools.partial(
        pl.kernel,
        mesh=mesh,
        out_type=jax.ShapeDtypeStruct((TOTAL_ROWS, EDIM), jnp.float32),
        scratch_types=(
            [pltpu.VMEM((rows_per_w,), jnp.int32),
             pltpu.VMEM((POS_REPS * MAXLEN, EDIM), jnp.float32)]
            + [pltpu.VMEM((UROWS, EDIM), jnp.float32) for _ in range(NBUF)]
            + [pltpu.SemaphoreType.DMA for _ in range(2 * NBUF)]
        ),
        compiler_params=pltpu.CompilerParams(use_tc_tiling_on_sc=False),
    )
    def sc_kernel(table_hbm, idx_hbm, pos_hbm, out_hbm, idx_v, pos_v, *rest):
        bufs = rest[:NBUF]
        gsem = rest[NBUF:2 * NBUF]
        ssem = rest[2 * NBUF:3 * NBUF]

        wid = lax.axis_index("s") * nc + lax.axis_index("c")
        base = wid * rows_per_w
        pltpu.sync_copy(idx_hbm.at[pl.ds(base, rows_per_w)], idx_v)
        pltpu.sync_copy(pos_hbm, pos_v)

        def start_g(u, b):
            for j in range(CHUNK):
                pltpu.async_copy(
                    table_hbm.at[idx_v.at[pl.ds(u * UROWS + j * GATHER, GATHER)]],
                    bufs[b].at[pl.ds(j * GATHER, GATHER)], gsem[b])

        def wait_g(b):
            for j in range(CHUNK):
                pltpu.make_async_copy(
                    table_hbm.at[idx_v.at[pl.ds(j * GATHER, GATHER)]],
                    bufs[b].at[pl.ds(j * GATHER, GATHER)], gsem[b]).wait()

        def add_pos(u, b):
            p0 = lax.rem(u * UROWS, MAXLEN)

            def row_add(r, c):
                p = p0 + r
                for h in range(EDIM // 16):
                    sl = pl.ds(h * 16, 16)
                    bufs[b][r, sl] = bufs[b][r, sl] + pos_v[p, sl]
                return c

            lax.fori_loop(0, UROWS, row_add, 0, unroll=4)

        def start_s(u, b):
            pltpu.async_copy(
                bufs[b], out_hbm.at[pl.ds(base + u * UROWS, UROWS)], ssem[b])

        def wait_s(b):
            pltpu.make_async_copy(
                bufs[b], out_hbm.at[pl.ds(base, UROWS)], ssem[b]).wait()

        @pl.loop(0, padded_steps, step=NBUF)
        def step_loop(s):
            for b in range(NBUF):
                u = s + b

                @pl.when(jnp.logical_and(u >= NBUF, u < units))
                def _():
                    wait_s(b)

                @pl.when(u < units)
                def _():
                    start_g(u, b)

                b1 = (b - GLAG) % NBUF
                u1 = u - GLAG

                @pl.when(jnp.logical_and(u1 >= 0, u1 < units))
                def _():
                    wait_g(b1)
                    add_pos(u1, b1)
                    start_s(u1, b1)

        for u in range(units - NBUF, units):
            wait_s(u % NBUF)

    return sc_kernel


_SC_KERNEL = _build_sc_kernel()


@jax.jit
def kernel(x, token_table, pos_table):
    xf = x.reshape(-1).astype(jnp.int32)
    pos_rep = jnp.tile(pos_table, (POS_REPS, 1))
    out = _SC_KERNEL(token_table, xf, pos_rep)
    return out.reshape(BATCH, MAXLEN, EDIM)
